# Initial kernel scaffold; baseline (speedup 1.0000x reference)
import functools
import jax
import jax.numpy as jnp
from jax import lax
from jax.experimental import pallas as pl
from jax.experimental.pallas import tpu as pltpu
from jax.experimental.pallas import tpu_sc as plsc

N = 10000
E = 320000
F_IN = 128
HID = 16
HEADS = 8
OUT = 40

NC = 2
NS = 16
NW = NC * NS
K = 200
EW = E // NW
NBLK = EW // K

_SLOPE = 0.2


def _leaky(v):
    return jnp.where(v >= 0.0, v, _SLOPE * v)


def _k1_body(x_ref, w1_ref, as_ref, ad_ref,
             th_ref, tas_ref, tad_ref, cvec_ref, initn_ref, initd_ref):
    h = jnp.dot(x_ref[...], w1_ref[...], preferred_element_type=jnp.float32)
    th_ref[...] = h
    ii = lax.broadcasted_iota(jnp.int32, (HEADS * HID, HEADS), 0)
    jj = lax.broadcasted_iota(jnp.int32, (HEADS * HID, HEADS), 1)
    S = jnp.where(ii // HID == jj, 1.0, 0.0)
    a_src = jnp.dot(h * as_ref[...], S, preferred_element_type=jnp.float32)
    a_dst = jnp.dot(h * ad_ref[...], S, preferred_element_type=jnp.float32)
    pad8 = jnp.zeros((N, 8), jnp.float32)
    tas_ref[...] = jnp.concatenate([a_src, pad8], axis=1)
    tad_ref[...] = jnp.concatenate([a_dst, pad8], axis=1)
    C = _leaky(jnp.max(a_src, axis=0) + jnp.max(a_dst, axis=0))
    cvec_ref[...] = jnp.concatenate([C[None, :], jnp.zeros((1, 8), jnp.float32)],
                                    axis=1)
    ex_self = jnp.exp(_leaky(a_src + a_dst) - C[None, :])
    ex_rep = jnp.dot(ex_self, S.T, preferred_element_type=jnp.float32)
    initn_ref[0] = h * ex_rep
    initn_ref[1] = jnp.zeros((N, HEADS * HID), jnp.float32)
    initd_ref[0] = jnp.concatenate([ex_self, pad8], axis=1)
    initd_ref[1] = jnp.zeros((N, 16), jnp.float32)


_k1 = pl.pallas_call(
    _k1_body,
    out_shape=[
        jax.ShapeDtypeStruct((N, HEADS * HID), jnp.float32),
        jax.ShapeDtypeStruct((N, 16), jnp.float32),
        jax.ShapeDtypeStruct((N, 16), jnp.float32),
        jax.ShapeDtypeStruct((1, 16), jnp.float32),
        jax.ShapeDtypeStruct((NC, N, HEADS * HID), jnp.float32),
        jax.ShapeDtypeStruct((NC, N, 16), jnp.float32),
    ],
)


def _sc1_body(th_hbm, tas_hbm, tad_hbm, cvec_hbm, initn_hbm, initd_hbm,
              src_hbm, dst_hbm, numo_hbm, deno_hbm,
              sidx, didx, hrow, asr, adr, cv, numS, denS):
    c = lax.axis_index("c")
    s = lax.axis_index("s")

    @pl.when(s == 0)
    def _():
        pltpu.sync_copy(initn_hbm.at[c], numS)
        pltpu.sync_copy(initd_hbm.at[c], denS)

    pltpu.sync_copy(cvec_hbm, cv)
    plsc.subcore_barrier()
    cvv = cv[0]

    @pl.loop(0, NBLK)
    def _blk(b):
        pltpu.sync_copy(src_hbm.at[c, s, b], sidx)
        pltpu.sync_copy(dst_hbm.at[c, s, b], didx)
        pltpu.sync_copy(th_hbm.at[sidx], hrow)
        pltpu.sync_copy(tas_hbm.at[sidx], asr)
        pltpu.sync_copy(tad_hbm.at[didx], adr)

        @pl.loop(0, K)
        def _edge(e):
            ee = asr[e] + adr[e]
            ee = jnp.where(ee >= 0.0, ee, _SLOPE * ee)
            ex = jnp.exp(ee - cvv)
            adr[e] = ex
            row_i = jnp.full((16,), e, jnp.int32)
            for hh in range(HEADS):
                exh = plsc.load_gather(adr, [row_i, jnp.full((16,), hh, jnp.int32)])
                hrow[e, pl.ds(HID * hh, HID)] = hrow[e, pl.ds(HID * hh, HID)] * exh

        pltpu.sync_copy(hrow, numS.at[didx], add=True)
        pltpu.sync_copy(adr, denS.at[didx], add=True)

    plsc.subcore_barrier()

    @pl.when(s == 0)
    def _():
        pltpu.sync_copy(numS, numo_hbm.at[c])
        pltpu.sync_copy(denS, deno_hbm.at[c])


_sc1 = functools.partial(
    pl.kernel,
    out_type=[
        jax.ShapeDtypeStruct((NC, N, HEADS * HID), jnp.float32),
        jax.ShapeDtypeStruct((NC, N, 16), jnp.float32),
    ],
    mesh=plsc.VectorSubcoreMesh(core_axis_name="c", subcore_axis_name="s"),
    scratch_types=[
        pltpu.VMEM((K,), jnp.int32),
        pltpu.VMEM((K,), jnp.int32),
        pltpu.VMEM((K, HEADS * HID), jnp.float32),
        pltpu.VMEM((K, 16), jnp.float32),
        pltpu.VMEM((K, 16), jnp.float32),
        pltpu.VMEM((1, 16), jnp.float32),
        pltpu.VMEM_SHARED((N, HEADS * HID), jnp.float32),
        pltpu.VMEM_SHARED((N, 16), jnp.float32),
    ],
)(_sc1_body)


def _k2_body(np_ref, dp_ref, b1_ref, w2_ref, as2_ref, ad2_ref,
             t2_ref, tad2_ref, cvec2_ref, initn2_ref):
    num = np_ref[0] + np_ref[1]
    den = dp_ref[0, :, 0:HEADS] + dp_ref[1, :, 0:HEADS]
    ii = lax.broadcasted_iota(jnp.int32, (HEADS, HEADS * HID), 0)
    jj = lax.broadcasted_iota(jnp.int32, (HEADS, HEADS * HID), 1)
    ST = jnp.where(jj // HID == ii, 1.0, 0.0)
    den_rep = jnp.dot(den, ST, preferred_element_type=jnp.float32)
    out1 = num / (den_rep + 1e-16) + b1_ref[...]
    h1 = jnp.maximum(out1, 0.0)
    h2 = jnp.dot(h1, w2_ref[...], preferred_element_type=jnp.float32)
    a2s = jnp.sum(h2 * as2_ref[...], axis=1, keepdims=True)
    a2d = jnp.sum(h2 * ad2_ref[...], axis=1, keepdims=True)
    C2 = _leaky(jnp.max(a2s) + jnp.max(a2d))
    ex2 = jnp.exp(_leaky(a2s + a2d) - C2)
    one = jnp.ones((N, 1), jnp.float32)
    z6 = jnp.zeros((N, 6), jnp.float32)
    t2_ref[...] = jnp.concatenate([h2, one, a2s, z6], axis=1)
    tad2_ref[...] = jnp.concatenate(
        [jnp.zeros((N, 9), jnp.float32), a2d, jnp.zeros((N, 6), jnp.float32)],
        axis=1)
    cvec2_ref[...] = jnp.full((1, 16), C2, jnp.float32)
    initn2_ref[0] = jnp.concatenate(
        [h2 * ex2, ex2, jnp.zeros((N, 7), jnp.float32)], axis=1)
    initn2_ref[1] = jnp.zeros((N, 48), jnp.float32)


_k2 = pl.pallas_call(
    _k2_body,
    out_shape=[
        jax.ShapeDtypeStruct((N, 48), jnp.float32),
        jax.ShapeDtypeStruct((N, 16), jnp.float32),
        jax.ShapeDtypeStruct((1, 16), jnp.float32),
        jax.ShapeDtypeStruct((NC, N, 48), jnp.float32),
    ],
)


def _sc2_body(t2_hbm, tad2_hbm, cvec2_hbm, initn2_hbm, src_hbm, dst_hbm,
              num2o_hbm, sidx, didx, hr2, adr2, cv2, n2S):
    c = lax.axis_index("c")
    s = lax.axis_index("s")

    @pl.when(s == 0)
    def _():
        pltpu.sync_copy(initn2_hbm.at[c], n2S)

    pltpu.sync_copy(cvec2_hbm, cv2)
    plsc.subcore_barrier()
    cvv = cv2[0]

    @pl.loop(0, NBLK)
    def _blk(b):
        pltpu.sync_copy(src_hbm.at[c, s, b], sidx)
        pltpu.sync_copy(dst_hbm.at[c, s, b], didx)
        pltpu.sync_copy(t2_hbm.at[sidx], hr2)
        pltpu.sync_copy(tad2_hbm.at[didx], adr2)

        @pl.loop(0, K)
        def _edge(e):
            ee = hr2[e, pl.ds(32, 16)] + adr2[e]
            ee = jnp.where(ee >= 0.0, ee, _SLOPE * ee)
            ex = jnp.exp(ee - cvv)
            adr2[e] = ex
            row_i = jnp.full((16,), e, jnp.int32)
            exs = plsc.load_gather(adr2, [row_i, jnp.full((16,), 9, jnp.int32)])
            for r in range(3):
                hr2[e, pl.ds(16 * r, 16)] = hr2[e, pl.ds(16 * r, 16)] * exs

        pltpu.sync_copy(hr2, n2S.at[didx], add=True)

    plsc.subcore_barrier()

    @pl.when(s == 0)
    def _():
        pltpu.sync_copy(n2S, num2o_hbm.at[c])


_sc2 = functools.partial(
    pl.kernel,
    out_type=jax.ShapeDtypeStruct((NC, N, 48), jnp.float32),
    mesh=plsc.VectorSubcoreMesh(core_axis_name="c", subcore_axis_name="s"),
    scratch_types=[
        pltpu.VMEM((K,), jnp.int32),
        pltpu.VMEM((K,), jnp.int32),
        pltpu.VMEM((K, 48), jnp.float32),
        pltpu.VMEM((K, 16), jnp.float32),
        pltpu.VMEM((1, 16), jnp.float32),
        pltpu.VMEM_SHARED((N, 48), jnp.float32),
    ],
)(_sc2_body)


def _k3_body(p_ref, b2_ref, out_ref):
    num2 = p_ref[0] + p_ref[1]
    o = num2[:, 0:OUT] / (num2[:, OUT:OUT + 1] + 1e-16) + b2_ref[...]
    m = jnp.max(o, axis=1, keepdims=True)
    z = o - m
    out_ref[...] = z - jnp.log(jnp.sum(jnp.exp(z), axis=1, keepdims=True))


_k3 = pl.pallas_call(
    _k3_body,
    out_shape=jax.ShapeDtypeStruct((N, OUT), jnp.float32),
)


@jax.jit
def kernel(x, edge_index, W1, att_src1, att_dst1, b1, W2, att_src2, att_dst2, b2):
    src = edge_index[0].reshape(NC, NS, NBLK, K)
    dst = edge_index[1].reshape(NC, NS, NBLK, K)
    as_flat = att_src1.reshape(1, HEADS * HID)
    ad_flat = att_dst1.reshape(1, HEADS * HID)

    th, tas, tad, cvec, initn, initd = _k1(x, W1, as_flat, ad_flat)
    nump, denp = _sc1(th, tas, tad, cvec, initn, initd, src, dst)
    t2, tad2, cvec2, initn2 = _k2(nump, denp, b1.reshape(1, HEADS * HID),
                                  W2, att_src2, att_dst2)
    num2p = _sc2(t2, tad2, cvec2, initn2, src, dst)
    return _k3(num2p, b2.reshape(1, OUT))


# speedup vs baseline: 39.0268x; 39.0268x over previous
import dataclasses
import functools
import jax
import jax.numpy as jnp
from jax import lax
from jax.experimental import pallas as pl
from jax.experimental.pallas import tpu as pltpu
from jax.experimental.pallas import tpu_sc as plsc

N = 10000
E = 320000
F_IN = 128
HID = 16
HEADS = 8
OUT = 40

NC = 2
NS = 16
NW = NC * NS
K = 200
EW = E // NW
NBLK = EW // K

_SLOPE = 0.2


def _leaky(v):
    return jnp.where(v >= 0.0, v, _SLOPE * v)


def _sc_compiler_params():
    cp = pltpu.CompilerParams()
    if "needs_layout_passes" in pltpu.CompilerParams.__dataclass_fields__:
        cp = dataclasses.replace(cp, needs_layout_passes=False)
    if "use_tc_tiling_on_sc" in pltpu.CompilerParams.__dataclass_fields__:
        cp = dataclasses.replace(cp, use_tc_tiling_on_sc=False)
    return cp


def _k1_body(x_ref, w1_ref, as_ref, ad_ref,
             th_ref, tas_ref, tad_ref, cvec_ref, initn_ref, initd_ref):
    h = jnp.dot(x_ref[...], w1_ref[...], preferred_element_type=jnp.float32)
    th_ref[...] = h
    ii = lax.broadcasted_iota(jnp.int32, (HEADS * HID, HEADS), 0)
    jj = lax.broadcasted_iota(jnp.int32, (HEADS * HID, HEADS), 1)
    S = jnp.where(ii // HID == jj, 1.0, 0.0)
    a_src = jnp.dot(h * as_ref[...], S, preferred_element_type=jnp.float32)
    a_dst = jnp.dot(h * ad_ref[...], S, preferred_element_type=jnp.float32)
    pad8 = jnp.zeros((N, 8), jnp.float32)
    tas_ref[...] = jnp.concatenate([a_src, pad8], axis=1)
    tad_ref[...] = jnp.concatenate([a_dst, pad8], axis=1)
    C = _leaky(jnp.max(a_src, axis=0) + jnp.max(a_dst, axis=0))
    cvec_ref[...] = jnp.concatenate([C[None, :], jnp.zeros((1, 8), jnp.float32)],
                                    axis=1)
    ex_self = jnp.exp(_leaky(a_src + a_dst) - C[None, :])
    ex_rep = jnp.dot(ex_self, S.T, preferred_element_type=jnp.float32)
    initn_ref[0] = h * ex_rep
    initn_ref[1] = jnp.zeros((N, HEADS * HID), jnp.float32)
    initd_ref[0] = jnp.concatenate([ex_self, pad8], axis=1)
    initd_ref[1] = jnp.zeros((N, 16), jnp.float32)


_k1 = pl.pallas_call(
    _k1_body,
    out_shape=[
        jax.ShapeDtypeStruct((N, HEADS * HID), jnp.float32),
        jax.ShapeDtypeStruct((N, 16), jnp.float32),
        jax.ShapeDtypeStruct((N, 16), jnp.float32),
        jax.ShapeDtypeStruct((1, 16), jnp.float32),
        jax.ShapeDtypeStruct((NC, N, HEADS * HID), jnp.float32),
        jax.ShapeDtypeStruct((NC, N, 16), jnp.float32),
    ],
)


def _sc1_body(th_hbm, tas_hbm, tad_hbm, cvec_hbm, initn_hbm, initd_hbm,
              src_hbm, dst_hbm, numo_hbm, deno_hbm,
              sidx, didx, hrow, asr, adr, cv, numS, denS):
    c = lax.axis_index("c")
    s = lax.axis_index("s")

    @pl.when(s == 0)
    def _():
        pltpu.sync_copy(initn_hbm.at[c], numS)
        pltpu.sync_copy(initd_hbm.at[c], denS)

    pltpu.sync_copy(cvec_hbm, cv)
    plsc.subcore_barrier()
    cvv = cv[0]

    @pl.loop(0, NBLK)
    def _blk(b):
        pltpu.sync_copy(src_hbm.at[c, s, b], sidx)
        pltpu.sync_copy(dst_hbm.at[c, s, b], didx)
        pltpu.sync_copy(th_hbm.at[sidx], hrow)
        pltpu.sync_copy(tas_hbm.at[sidx], asr)
        pltpu.sync_copy(tad_hbm.at[didx], adr)

        @pl.loop(0, K)
        def _edge(e):
            ee = asr[e] + adr[e]
            ee = jnp.where(ee >= 0.0, ee, _SLOPE * ee)
            ex = jnp.exp(ee - cvv)
            adr[e] = ex
            row_i = jnp.full((16,), e, jnp.int32)
            for hh in range(HEADS):
                exh = plsc.load_gather(adr, [row_i, jnp.full((16,), hh, jnp.int32)])
                hrow[e, pl.ds(HID * hh, HID)] = hrow[e, pl.ds(HID * hh, HID)] * exh

        pltpu.sync_copy(hrow, numS.at[didx], add=True)
        pltpu.sync_copy(adr, denS.at[didx], add=True)

    plsc.subcore_barrier()

    @pl.when(s == 0)
    def _():
        pltpu.sync_copy(numS, numo_hbm.at[c])
        pltpu.sync_copy(denS, deno_hbm.at[c])


@functools.cache
def _sc1():
    return pl.kernel(
        _sc1_body,
        out_type=[
            jax.ShapeDtypeStruct((NC, N, HEADS * HID), jnp.float32),
            jax.ShapeDtypeStruct((NC, N, 16), jnp.float32),
        ],
        mesh=plsc.VectorSubcoreMesh(core_axis_name="c", subcore_axis_name="s",
                                    num_cores=NC, num_subcores=NS),
        compiler_params=_sc_compiler_params(),
        scratch_types=[
            pltpu.VMEM((K,), jnp.int32),
            pltpu.VMEM((K,), jnp.int32),
            pltpu.VMEM((K, HEADS * HID), jnp.float32),
            pltpu.VMEM((K, 16), jnp.float32),
            pltpu.VMEM((K, 16), jnp.float32),
            pltpu.VMEM((1, 16), jnp.float32),
            pltpu.VMEM_SHARED((N, HEADS * HID), jnp.float32),
            pltpu.VMEM_SHARED((N, 16), jnp.float32),
        ],
    )


def _k2_body(np_ref, dp_ref, b1_ref, w2_ref, as2_ref, ad2_ref,
             t2_ref, tad2_ref, cvec2_ref, initn2_ref):
    num = np_ref[0] + np_ref[1]
    den = dp_ref[0, :, 0:HEADS] + dp_ref[1, :, 0:HEADS]
    ii = lax.broadcasted_iota(jnp.int32, (HEADS, HEADS * HID), 0)
    jj = lax.broadcasted_iota(jnp.int32, (HEADS, HEADS * HID), 1)
    ST = jnp.where(jj // HID == ii, 1.0, 0.0)
    den_rep = jnp.dot(den, ST, preferred_element_type=jnp.float32)
    out1 = num / (den_rep + 1e-16) + b1_ref[...]
    h1 = jnp.maximum(out1, 0.0)
    h2 = jnp.dot(h1, w2_ref[...], preferred_element_type=jnp.float32)
    a2s = jnp.sum(h2 * as2_ref[...], axis=1, keepdims=True)
    a2d = jnp.sum(h2 * ad2_ref[...], axis=1, keepdims=True)
    C2 = _leaky(jnp.max(a2s) + jnp.max(a2d))
    ex2 = jnp.exp(_leaky(a2s + a2d) - C2)
    one = jnp.ones((N, 1), jnp.float32)
    z6 = jnp.zeros((N, 6), jnp.float32)
    t2_ref[...] = jnp.concatenate([h2, one, a2s, z6], axis=1)
    tad2_ref[...] = jnp.concatenate(
        [jnp.zeros((N, 9), jnp.float32), a2d, jnp.zeros((N, 6), jnp.float32)],
        axis=1)
    cvec2_ref[...] = jnp.full((1, 16), C2, jnp.float32)
    initn2_ref[0] = jnp.concatenate(
        [h2 * ex2, ex2, jnp.zeros((N, 7), jnp.float32)], axis=1)
    initn2_ref[1] = jnp.zeros((N, 48), jnp.float32)


_k2 = pl.pallas_call(
    _k2_body,
    out_shape=[
        jax.ShapeDtypeStruct((N, 48), jnp.float32),
        jax.ShapeDtypeStruct((N, 16), jnp.float32),
        jax.ShapeDtypeStruct((1, 16), jnp.float32),
        jax.ShapeDtypeStruct((NC, N, 48), jnp.float32),
    ],
)


def _sc2_body(t2_hbm, tad2_hbm, cvec2_hbm, initn2_hbm, src_hbm, dst_hbm,
              num2o_hbm, sidx, didx, hr2, adr2, cv2, n2S):
    c = lax.axis_index("c")
    s = lax.axis_index("s")

    @pl.when(s == 0)
    def _():
        pltpu.sync_copy(initn2_hbm.at[c], n2S)

    pltpu.sync_copy(cvec2_hbm, cv2)
    plsc.subcore_barrier()
    cvv = cv2[0]

    @pl.loop(0, NBLK)
    def _blk(b):
        pltpu.sync_copy(src_hbm.at[c, s, b], sidx)
        pltpu.sync_copy(dst_hbm.at[c, s, b], didx)
        pltpu.sync_copy(t2_hbm.at[sidx], hr2)
        pltpu.sync_copy(tad2_hbm.at[didx], adr2)

        @pl.loop(0, K)
        def _edge(e):
            ee = hr2[e, pl.ds(32, 16)] + adr2[e]
            ee = jnp.where(ee >= 0.0, ee, _SLOPE * ee)
            ex = jnp.exp(ee - cvv)
            adr2[e] = ex
            row_i = jnp.full((16,), e, jnp.int32)
            exs = plsc.load_gather(adr2, [row_i, jnp.full((16,), 9, jnp.int32)])
            for r in range(3):
                hr2[e, pl.ds(16 * r, 16)] = hr2[e, pl.ds(16 * r, 16)] * exs

        pltpu.sync_copy(hr2, n2S.at[didx], add=True)

    plsc.subcore_barrier()

    @pl.when(s == 0)
    def _():
        pltpu.sync_copy(n2S, num2o_hbm.at[c])


@functools.cache
def _sc2():
    return pl.kernel(
        _sc2_body,
        out_type=jax.ShapeDtypeStruct((NC, N, 48), jnp.float32),
        mesh=plsc.VectorSubcoreMesh(core_axis_name="c", subcore_axis_name="s",
                                    num_cores=NC, num_subcores=NS),
        compiler_params=_sc_compiler_params(),
        scratch_types=[
            pltpu.VMEM((K,), jnp.int32),
            pltpu.VMEM((K,), jnp.int32),
            pltpu.VMEM((K, 48), jnp.float32),
            pltpu.VMEM((K, 16), jnp.float32),
            pltpu.VMEM((1, 16), jnp.float32),
            pltpu.VMEM_SHARED((N, 48), jnp.float32),
        ],
    )


def _k3_body(p_ref, b2_ref, out_ref):
    num2 = p_ref[0] + p_ref[1]
    o = num2[:, 0:OUT] / (num2[:, OUT:OUT + 1] + 1e-16) + b2_ref[...]
    m = jnp.max(o, axis=1, keepdims=True)
    z = o - m
    out_ref[...] = z - jnp.log(jnp.sum(jnp.exp(z), axis=1, keepdims=True))


_k3 = pl.pallas_call(
    _k3_body,
    out_shape=jax.ShapeDtypeStruct((N, OUT), jnp.float32),
)


@jax.jit
def kernel(x, edge_index, W1, att_src1, att_dst1, b1, W2, att_src2, att_dst2, b2):
    src = edge_index[0].reshape(NC, NS, NBLK, K)
    dst = edge_index[1].reshape(NC, NS, NBLK, K)
    as_flat = att_src1.reshape(1, HEADS * HID)
    ad_flat = att_dst1.reshape(1, HEADS * HID)

    th, tas, tad, cvec, initn, initd = _k1(x, W1, as_flat, ad_flat)
    nump, denp = _sc1()(th, tas, tad, cvec, initn, initd, src, dst)
    t2, tad2, cvec2, initn2 = _k2(nump, denp, b1.reshape(1, HEADS * HID),
                                  W2, att_src2, att_dst2)
    num2p = _sc2()(t2, tad2, cvec2, initn2, src, dst)
    return _k3(num2p, b2.reshape(1, OUT))


# speedup vs baseline: 79.1242x; 2.0274x over previous
import dataclasses
import functools
import jax
import jax.numpy as jnp
from jax import lax
from jax.experimental import pallas as pl
from jax.experimental.pallas import tpu as pltpu
from jax.experimental.pallas import tpu_sc as plsc

N = 10000
E = 320000
F_IN = 128
HID = 16
HEADS = 8
OUT = 40

NC = 2
NS = 16
NW = NC * NS
K = 200
EW = E // NW
NBLK = EW // K

_SLOPE = 0.2


def _leaky(v):
    return jnp.where(v >= 0.0, v, _SLOPE * v)


def _sc_compiler_params():
    cp = pltpu.CompilerParams()
    if "needs_layout_passes" in pltpu.CompilerParams.__dataclass_fields__:
        cp = dataclasses.replace(cp, needs_layout_passes=False)
    if "use_tc_tiling_on_sc" in pltpu.CompilerParams.__dataclass_fields__:
        cp = dataclasses.replace(cp, use_tc_tiling_on_sc=False)
    return cp


def _k1_body(x_ref, w1_ref, as_ref, ad_ref,
             th_ref, tas_ref, tad_ref, cvec_ref, initn_ref, initd_ref):
    h = jnp.dot(x_ref[...], w1_ref[...], preferred_element_type=jnp.float32)
    th_ref[...] = h
    ii = lax.broadcasted_iota(jnp.int32, (HEADS * HID, HEADS), 0)
    jj = lax.broadcasted_iota(jnp.int32, (HEADS * HID, HEADS), 1)
    S = jnp.where(ii // HID == jj, 1.0, 0.0)
    a_src = jnp.dot(h * as_ref[...], S, preferred_element_type=jnp.float32)
    a_dst = jnp.dot(h * ad_ref[...], S, preferred_element_type=jnp.float32)
    pad8 = jnp.zeros((N, 8), jnp.float32)
    tas_ref[...] = jnp.concatenate([a_src, pad8], axis=1)
    tad_ref[...] = jnp.concatenate([a_dst, pad8], axis=1)
    C = _leaky(jnp.max(a_src, axis=0) + jnp.max(a_dst, axis=0))
    cvec_ref[...] = jnp.concatenate([C[None, :], jnp.zeros((1, 8), jnp.float32)],
                                    axis=1)
    ex_self = jnp.exp(_leaky(a_src + a_dst) - C[None, :])
    ex_rep = jnp.dot(ex_self, S.T, preferred_element_type=jnp.float32)
    initn_ref[0] = h * ex_rep
    initn_ref[1] = jnp.zeros((N, HEADS * HID), jnp.float32)
    initd_ref[0] = jnp.concatenate([ex_self, pad8], axis=1)
    initd_ref[1] = jnp.zeros((N, 16), jnp.float32)


_k1 = pl.pallas_call(
    _k1_body,
    out_shape=[
        jax.ShapeDtypeStruct((N, HEADS * HID), jnp.float32),
        jax.ShapeDtypeStruct((N, 16), jnp.float32),
        jax.ShapeDtypeStruct((N, 16), jnp.float32),
        jax.ShapeDtypeStruct((1, 16), jnp.float32),
        jax.ShapeDtypeStruct((NC, N, HEADS * HID), jnp.float32),
        jax.ShapeDtypeStruct((NC, N, 16), jnp.float32),
    ],
)


def _sc1_body(th_hbm, tas_hbm, tad_hbm, cvec_hbm, initn_hbm, initd_hbm,
              src_hbm, dst_hbm, numo_hbm, deno_hbm,
              sidx, didx, hrow, asr, adr, cv, numS, denS):
    c = lax.axis_index("c")
    s = lax.axis_index("s")

    @pl.when(s == 0)
    def _():
        pltpu.sync_copy(initn_hbm.at[c], numS)
        pltpu.sync_copy(initd_hbm.at[c], denS)

    pltpu.sync_copy(cvec_hbm, cv)
    plsc.subcore_barrier()
    cvv = cv[0]

    @pl.loop(0, NBLK)
    def _blk(b):
        pltpu.sync_copy(src_hbm.at[c, s, b], sidx)
        pltpu.sync_copy(dst_hbm.at[c, s, b], didx)
        pltpu.sync_copy(th_hbm.at[sidx], hrow)
        pltpu.sync_copy(tas_hbm.at[sidx], asr)
        pltpu.sync_copy(tad_hbm.at[didx], adr)

        @plsc.parallel_loop(0, K, unroll=4)
        def _edge(e):
            ee = asr[e] + adr[e]
            ee = jnp.where(ee >= 0.0, ee, _SLOPE * ee)
            ex = jnp.exp(ee - cvv)
            adr[e] = ex
            for hh in range(HEADS):
                exh = ex.at[jnp.full((16,), hh, jnp.int32)].get(
                    mode="promise_in_bounds")
                hrow[e, pl.ds(HID * hh, HID)] = hrow[e, pl.ds(HID * hh, HID)] * exh

        pltpu.sync_copy(hrow, numS.at[didx], add=True)
        pltpu.sync_copy(adr, denS.at[didx], add=True)

    plsc.subcore_barrier()

    @pl.when(s == 0)
    def _():
        pltpu.sync_copy(numS, numo_hbm.at[c])
        pltpu.sync_copy(denS, deno_hbm.at[c])


@functools.cache
def _sc1():
    return pl.kernel(
        _sc1_body,
        out_type=[
            jax.ShapeDtypeStruct((NC, N, HEADS * HID), jnp.float32),
            jax.ShapeDtypeStruct((NC, N, 16), jnp.float32),
        ],
        mesh=plsc.VectorSubcoreMesh(core_axis_name="c", subcore_axis_name="s",
                                    num_cores=NC, num_subcores=NS),
        compiler_params=_sc_compiler_params(),
        scratch_types=[
            pltpu.VMEM((K,), jnp.int32),
            pltpu.VMEM((K,), jnp.int32),
            pltpu.VMEM((K, HEADS * HID), jnp.float32),
            pltpu.VMEM((K, 16), jnp.float32),
            pltpu.VMEM((K, 16), jnp.float32),
            pltpu.VMEM((1, 16), jnp.float32),
            pltpu.VMEM_SHARED((N, HEADS * HID), jnp.float32),
            pltpu.VMEM_SHARED((N, 16), jnp.float32),
        ],
    )


def _k2_body(np_ref, dp_ref, b1_ref, w2_ref, as2_ref, ad2_ref,
             t2_ref, tad2_ref, cvec2_ref, initn2_ref):
    num = np_ref[0] + np_ref[1]
    den = dp_ref[0, :, 0:HEADS] + dp_ref[1, :, 0:HEADS]
    ii = lax.broadcasted_iota(jnp.int32, (HEADS, HEADS * HID), 0)
    jj = lax.broadcasted_iota(jnp.int32, (HEADS, HEADS * HID), 1)
    ST = jnp.where(jj // HID == ii, 1.0, 0.0)
    den_rep = jnp.dot(den, ST, preferred_element_type=jnp.float32)
    out1 = num / (den_rep + 1e-16) + b1_ref[...]
    h1 = jnp.maximum(out1, 0.0)
    h2 = jnp.dot(h1, w2_ref[...], preferred_element_type=jnp.float32)
    a2s = jnp.sum(h2 * as2_ref[...], axis=1, keepdims=True)
    a2d = jnp.sum(h2 * ad2_ref[...], axis=1, keepdims=True)
    C2 = _leaky(jnp.max(a2s) + jnp.max(a2d))
    ex2 = jnp.exp(_leaky(a2s + a2d) - C2)
    one = jnp.ones((N, 1), jnp.float32)
    z6 = jnp.zeros((N, 6), jnp.float32)
    t2_ref[...] = jnp.concatenate([h2, one, a2s, z6], axis=1)
    tad2_ref[...] = jnp.concatenate(
        [jnp.zeros((N, 9), jnp.float32), a2d, jnp.zeros((N, 6), jnp.float32)],
        axis=1)
    cvec2_ref[...] = jnp.full((1, 16), C2, jnp.float32)
    initn2_ref[0] = jnp.concatenate(
        [h2 * ex2, ex2, jnp.zeros((N, 7), jnp.float32)], axis=1)
    initn2_ref[1] = jnp.zeros((N, 48), jnp.float32)


_k2 = pl.pallas_call(
    _k2_body,
    out_shape=[
        jax.ShapeDtypeStruct((N, 48), jnp.float32),
        jax.ShapeDtypeStruct((N, 16), jnp.float32),
        jax.ShapeDtypeStruct((1, 16), jnp.float32),
        jax.ShapeDtypeStruct((NC, N, 48), jnp.float32),
    ],
)


def _sc2_body(t2_hbm, tad2_hbm, cvec2_hbm, initn2_hbm, src_hbm, dst_hbm,
              num2o_hbm, sidx, didx, hr2, adr2, cv2, n2S):
    c = lax.axis_index("c")
    s = lax.axis_index("s")

    @pl.when(s == 0)
    def _():
        pltpu.sync_copy(initn2_hbm.at[c], n2S)

    pltpu.sync_copy(cvec2_hbm, cv2)
    plsc.subcore_barrier()
    cvv = cv2[0]

    @pl.loop(0, NBLK)
    def _blk(b):
        pltpu.sync_copy(src_hbm.at[c, s, b], sidx)
        pltpu.sync_copy(dst_hbm.at[c, s, b], didx)
        pltpu.sync_copy(t2_hbm.at[sidx], hr2)
        pltpu.sync_copy(tad2_hbm.at[didx], adr2)

        @plsc.parallel_loop(0, K, unroll=4)
        def _edge(e):
            ee = hr2[e, pl.ds(32, 16)] + adr2[e]
            ee = jnp.where(ee >= 0.0, ee, _SLOPE * ee)
            ex = jnp.exp(ee - cvv)
            exs = ex.at[jnp.full((16,), 9, jnp.int32)].get(
                mode="promise_in_bounds")
            for r in range(2):
                hr2[e, pl.ds(16 * r, 16)] = hr2[e, pl.ds(16 * r, 16)] * exs
            hr2[e, pl.ds(32, 16)] = hr2[e, pl.ds(32, 16)] * exs

        pltpu.sync_copy(hr2, n2S.at[didx], add=True)

    plsc.subcore_barrier()

    @pl.when(s == 0)
    def _():
        pltpu.sync_copy(n2S, num2o_hbm.at[c])


@functools.cache
def _sc2():
    return pl.kernel(
        _sc2_body,
        out_type=jax.ShapeDtypeStruct((NC, N, 48), jnp.float32),
        mesh=plsc.VectorSubcoreMesh(core_axis_name="c", subcore_axis_name="s",
                                    num_cores=NC, num_subcores=NS),
        compiler_params=_sc_compiler_params(),
        scratch_types=[
            pltpu.VMEM((K,), jnp.int32),
            pltpu.VMEM((K,), jnp.int32),
            pltpu.VMEM((K, 48), jnp.float32),
            pltpu.VMEM((K, 16), jnp.float32),
            pltpu.VMEM((1, 16), jnp.float32),
            pltpu.VMEM_SHARED((N, 48), jnp.float32),
        ],
    )


def _k3_body(p_ref, b2_ref, out_ref):
    num2 = p_ref[0] + p_ref[1]
    o = num2[:, 0:OUT] / (num2[:, OUT:OUT + 1] + 1e-16) + b2_ref[...]
    m = jnp.max(o, axis=1, keepdims=True)
    z = o - m
    out_ref[...] = z - jnp.log(jnp.sum(jnp.exp(z), axis=1, keepdims=True))


_k3 = pl.pallas_call(
    _k3_body,
    out_shape=jax.ShapeDtypeStruct((N, OUT), jnp.float32),
)


@jax.jit
def kernel(x, edge_index, W1, att_src1, att_dst1, b1, W2, att_src2, att_dst2, b2):
    src = edge_index[0].reshape(NC, NS, NBLK, K)
    dst = edge_index[1].reshape(NC, NS, NBLK, K)
    as_flat = att_src1.reshape(1, HEADS * HID)
    ad_flat = att_dst1.reshape(1, HEADS * HID)

    th, tas, tad, cvec, initn, initd = _k1(x, W1, as_flat, ad_flat)
    nump, denp = _sc1()(th, tas, tad, cvec, initn, initd, src, dst)
    t2, tad2, cvec2, initn2 = _k2(nump, denp, b1.reshape(1, HEADS * HID),
                                  W2, att_src2, att_dst2)
    num2p = _sc2()(t2, tad2, cvec2, initn2, src, dst)
    return _k3(num2p, b2.reshape(1, OUT))


# speedup vs baseline: 99.5575x; 1.2582x over previous
import dataclasses
import functools
import jax
import jax.numpy as jnp
from jax import lax
from jax.experimental import pallas as pl
from jax.experimental.pallas import tpu as pltpu
from jax.experimental.pallas import tpu_sc as plsc

N = 10000
E = 320000
F_IN = 128
HID = 16
HEADS = 8
OUT = 40

NC = 2
NS = 16
NW = NC * NS
EW = E // NW
G = 5

K1 = 40
NBLK1 = EW // K1
NGRP1 = NBLK1 // G

K2 = 200
NBLK2 = EW // K2
NGRP2 = NBLK2 // G

_SLOPE = 0.2


def _leaky(v):
    return jnp.where(v >= 0.0, v, _SLOPE * v)


def _sc_compiler_params():
    cp = pltpu.CompilerParams()
    if "needs_layout_passes" in pltpu.CompilerParams.__dataclass_fields__:
        cp = dataclasses.replace(cp, needs_layout_passes=False)
    if "use_tc_tiling_on_sc" in pltpu.CompilerParams.__dataclass_fields__:
        cp = dataclasses.replace(cp, use_tc_tiling_on_sc=False)
    return cp


def _k1_body(x_ref, w1_ref, as_ref, ad_ref,
             th_ref, tas_ref, tad_ref, cvec_ref, initn_ref, initd_ref):
    h = jnp.dot(x_ref[...], w1_ref[...], preferred_element_type=jnp.float32)
    th_ref[...] = h
    ii = lax.broadcasted_iota(jnp.int32, (HEADS * HID, HEADS), 0)
    jj = lax.broadcasted_iota(jnp.int32, (HEADS * HID, HEADS), 1)
    S = jnp.where(ii // HID == jj, 1.0, 0.0)
    a_src = jnp.dot(h * as_ref[...], S, preferred_element_type=jnp.float32)
    a_dst = jnp.dot(h * ad_ref[...], S, preferred_element_type=jnp.float32)
    pad8 = jnp.zeros((N, 8), jnp.float32)
    tas_ref[...] = jnp.concatenate([a_src, pad8], axis=1)
    tad_ref[...] = jnp.concatenate([a_dst, pad8], axis=1)
    C = _leaky(jnp.max(a_src, axis=0) + jnp.max(a_dst, axis=0))
    cvec_ref[...] = jnp.concatenate([C[None, :], jnp.zeros((1, 8), jnp.float32)],
                                    axis=1)
    ex_self = jnp.exp(_leaky(a_src + a_dst) - C[None, :])
    ex_rep = jnp.dot(ex_self, S.T, preferred_element_type=jnp.float32)
    initn_ref[0] = h * ex_rep
    initn_ref[1] = jnp.zeros((N, HEADS * HID), jnp.float32)
    initd_ref[0] = jnp.concatenate([ex_self, pad8], axis=1)
    initd_ref[1] = jnp.zeros((N, 16), jnp.float32)


_k1 = pl.pallas_call(
    _k1_body,
    out_shape=[
        jax.ShapeDtypeStruct((N, HEADS * HID), jnp.float32),
        jax.ShapeDtypeStruct((N, 16), jnp.float32),
        jax.ShapeDtypeStruct((N, 16), jnp.float32),
        jax.ShapeDtypeStruct((1, 16), jnp.float32),
        jax.ShapeDtypeStruct((NC, N, HEADS * HID), jnp.float32),
        jax.ShapeDtypeStruct((NC, N, 16), jnp.float32),
    ],
)


def _sc1_body(th_hbm, tas_hbm, tad_hbm, cvec_hbm, initn_hbm, initd_hbm,
              src_hbm, dst_hbm, numo_hbm, deno_hbm,
              sidx, didx, hrow, asr, adr, cv, numS, denS,
              g0, g1, g2, g3, g4, s0, s1, s2, s3, s4):
    c = lax.axis_index("c")
    s = lax.axis_index("s")
    gsem = (g0, g1, g2, g3, g4)
    ssem = (s0, s1, s2, s3, s4)

    @pl.when(s == 0)
    def _():
        pltpu.sync_copy(initn_hbm.at[c], numS)
        pltpu.sync_copy(initd_hbm.at[c], denS)

    pltpu.sync_copy(cvec_hbm, cv)
    plsc.subcore_barrier()
    cvv = cv[0]

    @pl.loop(0, NGRP1)
    def _grp(g):
        gd = []
        for j in range(G):
            b = g * G + j
            pltpu.sync_copy(src_hbm.at[c, s, b], sidx.at[j])
            pltpu.sync_copy(dst_hbm.at[c, s, b], didx.at[j])
            gd.append((
                pltpu.async_copy(th_hbm.at[sidx.at[j]], hrow.at[j], gsem[j]),
                pltpu.async_copy(tas_hbm.at[sidx.at[j]], asr.at[j], gsem[j]),
                pltpu.async_copy(tad_hbm.at[didx.at[j]], adr.at[j], gsem[j]),
            ))
        sd = []
        for j in range(G):
            for d in gd[j]:
                d.wait()
            hb = hrow.at[j]
            ab = asr.at[j]
            db = adr.at[j]

            @plsc.parallel_loop(0, K1, unroll=4)
            def _edge(e, hb=hb, ab=ab, db=db):
                ee = ab[e] + db[e]
                ee = jnp.where(ee >= 0.0, ee, _SLOPE * ee)
                ex = jnp.exp(ee - cvv)
                db[e] = ex
                for hh in range(HEADS):
                    exh = ex.at[jnp.full((16,), hh, jnp.int32)].get(
                        mode="promise_in_bounds")
                    hb[e, pl.ds(HID * hh, HID)] = (
                        hb[e, pl.ds(HID * hh, HID)] * exh)

            sd.append((
                pltpu.async_copy(hb, numS.at[didx.at[j]], ssem[j], add=True),
                pltpu.async_copy(db, denS.at[didx.at[j]], ssem[j], add=True),
            ))
        for pair in sd:
            for d in pair:
                d.wait()

    plsc.subcore_barrier()

    @pl.when(s == 0)
    def _():
        pltpu.sync_copy(numS, numo_hbm.at[c])
        pltpu.sync_copy(denS, deno_hbm.at[c])


@functools.cache
def _sc1():
    return pl.kernel(
        _sc1_body,
        out_type=[
            jax.ShapeDtypeStruct((NC, N, HEADS * HID), jnp.float32),
            jax.ShapeDtypeStruct((NC, N, 16), jnp.float32),
        ],
        mesh=plsc.VectorSubcoreMesh(core_axis_name="c", subcore_axis_name="s",
                                    num_cores=NC, num_subcores=NS),
        compiler_params=_sc_compiler_params(),
        scratch_types=[
            pltpu.VMEM((G, K1), jnp.int32),
            pltpu.VMEM((G, K1), jnp.int32),
            pltpu.VMEM((G, K1, HEADS * HID), jnp.float32),
            pltpu.VMEM((G, K1, 16), jnp.float32),
            pltpu.VMEM((G, K1, 16), jnp.float32),
            pltpu.VMEM((1, 16), jnp.float32),
            pltpu.VMEM_SHARED((N, HEADS * HID), jnp.float32),
            pltpu.VMEM_SHARED((N, 16), jnp.float32),
            pltpu.SemaphoreType.DMA,
            pltpu.SemaphoreType.DMA,
            pltpu.SemaphoreType.DMA,
            pltpu.SemaphoreType.DMA,
            pltpu.SemaphoreType.DMA,
            pltpu.SemaphoreType.DMA,
            pltpu.SemaphoreType.DMA,
            pltpu.SemaphoreType.DMA,
            pltpu.SemaphoreType.DMA,
            pltpu.SemaphoreType.DMA,
        ],
    )


def _k2_body(np_ref, dp_ref, b1_ref, w2_ref, as2_ref, ad2_ref,
             t2_ref, tad2_ref, cvec2_ref, initn2_ref):
    num = np_ref[0] + np_ref[1]
    den = dp_ref[0, :, 0:HEADS] + dp_ref[1, :, 0:HEADS]
    ii = lax.broadcasted_iota(jnp.int32, (HEADS, HEADS * HID), 0)
    jj = lax.broadcasted_iota(jnp.int32, (HEADS, HEADS * HID), 1)
    ST = jnp.where(jj // HID == ii, 1.0, 0.0)
    den_rep = jnp.dot(den, ST, preferred_element_type=jnp.float32)
    out1 = num / (den_rep + 1e-16) + b1_ref[...]
    h1 = jnp.maximum(out1, 0.0)
    h2 = jnp.dot(h1, w2_ref[...], preferred_element_type=jnp.float32)
    a2s = jnp.sum(h2 * as2_ref[...], axis=1, keepdims=True)
    a2d = jnp.sum(h2 * ad2_ref[...], axis=1, keepdims=True)
    C2 = _leaky(jnp.max(a2s) + jnp.max(a2d))
    ex2 = jnp.exp(_leaky(a2s + a2d) - C2)
    one = jnp.ones((N, 1), jnp.float32)
    z6 = jnp.zeros((N, 6), jnp.float32)
    t2_ref[...] = jnp.concatenate([h2, one, a2s, z6], axis=1)
    tad2_ref[...] = jnp.concatenate(
        [jnp.zeros((N, 9), jnp.float32), a2d, jnp.zeros((N, 6), jnp.float32)],
        axis=1)
    cvec2_ref[...] = jnp.full((1, 16), C2, jnp.float32)
    initn2_ref[0] = jnp.concatenate(
        [h2 * ex2, ex2, jnp.zeros((N, 7), jnp.float32)], axis=1)
    initn2_ref[1] = jnp.zeros((N, 48), jnp.float32)


_k2 = pl.pallas_call(
    _k2_body,
    out_shape=[
        jax.ShapeDtypeStruct((N, 48), jnp.float32),
        jax.ShapeDtypeStruct((N, 16), jnp.float32),
        jax.ShapeDtypeStruct((1, 16), jnp.float32),
        jax.ShapeDtypeStruct((NC, N, 48), jnp.float32),
    ],
)


def _sc2_body(t2_hbm, tad2_hbm, cvec2_hbm, initn2_hbm, src_hbm, dst_hbm,
              num2o_hbm, sidx, didx, hr2, adr2, cv2, n2S,
              g0, g1, g2, g3, g4, s0, s1, s2, s3, s4):
    c = lax.axis_index("c")
    s = lax.axis_index("s")
    gsem = (g0, g1, g2, g3, g4)
    ssem = (s0, s1, s2, s3, s4)

    @pl.when(s == 0)
    def _():
        pltpu.sync_copy(initn2_hbm.at[c], n2S)

    pltpu.sync_copy(cvec2_hbm, cv2)
    plsc.subcore_barrier()
    cvv = cv2[0]

    @pl.loop(0, NGRP2)
    def _grp(g):
        gd = []
        for j in range(G):
            b = g * G + j
            pltpu.sync_copy(src_hbm.at[c, s, b], sidx.at[j])
            pltpu.sync_copy(dst_hbm.at[c, s, b], didx.at[j])
            gd.append((
                pltpu.async_copy(t2_hbm.at[sidx.at[j]], hr2.at[j], gsem[j]),
                pltpu.async_copy(tad2_hbm.at[didx.at[j]], adr2.at[j], gsem[j]),
            ))
        sd = []
        for j in range(G):
            for d in gd[j]:
                d.wait()
            hb = hr2.at[j]
            db = adr2.at[j]

            @plsc.parallel_loop(0, K2, unroll=4)
            def _edge(e, hb=hb, db=db):
                ee = hb[e, pl.ds(32, 16)] + db[e]
                ee = jnp.where(ee >= 0.0, ee, _SLOPE * ee)
                ex = jnp.exp(ee - cvv)
                exs = ex.at[jnp.full((16,), 9, jnp.int32)].get(
                    mode="promise_in_bounds")
                for r in range(2):
                    hb[e, pl.ds(16 * r, 16)] = hb[e, pl.ds(16 * r, 16)] * exs
                hb[e, pl.ds(32, 16)] = hb[e, pl.ds(32, 16)] * exs

            sd.append(
                pltpu.async_copy(hb, n2S.at[didx.at[j]], ssem[j], add=True))
        for d in sd:
            d.wait()

    plsc.subcore_barrier()

    @pl.when(s == 0)
    def _():
        pltpu.sync_copy(n2S, num2o_hbm.at[c])


@functools.cache
def _sc2():
    return pl.kernel(
        _sc2_body,
        out_type=jax.ShapeDtypeStruct((NC, N, 48), jnp.float32),
        mesh=plsc.VectorSubcoreMesh(core_axis_name="c", subcore_axis_name="s",
                                    num_cores=NC, num_subcores=NS),
        compiler_params=_sc_compiler_params(),
        scratch_types=[
            pltpu.VMEM((G, K2), jnp.int32),
            pltpu.VMEM((G, K2), jnp.int32),
            pltpu.VMEM((G, K2, 48), jnp.float32),
            pltpu.VMEM((G, K2, 16), jnp.float32),
            pltpu.VMEM((1, 16), jnp.float32),
            pltpu.VMEM_SHARED((N, 48), jnp.float32),
            pltpu.SemaphoreType.DMA,
            pltpu.SemaphoreType.DMA,
            pltpu.SemaphoreType.DMA,
            pltpu.SemaphoreType.DMA,
            pltpu.SemaphoreType.DMA,
            pltpu.SemaphoreType.DMA,
            pltpu.SemaphoreType.DMA,
            pltpu.SemaphoreType.DMA,
            pltpu.SemaphoreType.DMA,
            pltpu.SemaphoreType.DMA,
        ],
    )


def _k3_body(p_ref, b2_ref, out_ref):
    num2 = p_ref[0] + p_ref[1]
    o = num2[:, 0:OUT] / (num2[:, OUT:OUT + 1] + 1e-16) + b2_ref[...]
    m = jnp.max(o, axis=1, keepdims=True)
    z = o - m
    out_ref[...] = z - jnp.log(jnp.sum(jnp.exp(z), axis=1, keepdims=True))


_k3 = pl.pallas_call(
    _k3_body,
    out_shape=jax.ShapeDtypeStruct((N, OUT), jnp.float32),
)


@jax.jit
def kernel(x, edge_index, W1, att_src1, att_dst1, b1, W2, att_src2, att_dst2, b2):
    src1 = edge_index[0].reshape(NC, NS, NBLK1, K1)
    dst1 = edge_index[1].reshape(NC, NS, NBLK1, K1)
    src2 = edge_index[0].reshape(NC, NS, NBLK2, K2)
    dst2 = edge_index[1].reshape(NC, NS, NBLK2, K2)
    as_flat = att_src1.reshape(1, HEADS * HID)
    ad_flat = att_dst1.reshape(1, HEADS * HID)

    th, tas, tad, cvec, initn, initd = _k1(x, W1, as_flat, ad_flat)
    nump, denp = _sc1()(th, tas, tad, cvec, initn, initd, src1, dst1)
    t2, tad2, cvec2, initn2 = _k2(nump, denp, b1.reshape(1, HEADS * HID),
                                  W2, att_src2, att_dst2)
    num2p = _sc2()(t2, tad2, cvec2, initn2, src2, dst2)
    return _k3(num2p, b2.reshape(1, OUT))
